# Initial kernel scaffold; baseline (speedup 1.0000x reference)
import jax
import jax.numpy as jnp
from jax.experimental import pallas as pl


def kernel(x, edge_index, x0, W, b):
    raise NotImplementedError("write your pallas kernel here")


# speedup vs baseline: 11.6103x; 11.6103x over previous
import jax
import jax.numpy as jnp
from jax import lax
from jax.experimental import pallas as pl
from jax.experimental.pallas import tpu as pltpu
from jax.experimental.pallas import tpu_sc as plsc

N_NODES = 10000
N_EDGES = 320000
D = 128

NC = 2
NS = 16
L = 16

N_PAD = 10240
ROWS_PER_TILE = N_PAD // NS
GROUPS_PER_TILE = ROWS_PER_TILE // L

EC = 80
E_PER_TILE = N_EDGES // (NC * NS)
N_ECHUNK = E_PER_TILE // EC
DEG_PER_TILE = N_EDGES // NS
N_DCHUNK = DEG_PER_TILE // EC


def _rsqrt16(v):
    i = lax.bitcast_convert_type(v, jnp.int32)
    i = jnp.int32(0x5F3759DF) - lax.shift_right_logical(i, 1)
    y = lax.bitcast_convert_type(i, jnp.float32)
    half = v * 0.5
    for _ in range(3):
        y = y * (1.5 - half * y * y)
    return y


def _scale_rows(vb, dv, k):
    for r in range(L):
        bc = plsc.load_gather(dv, [jnp.full((L,), k * L + r, jnp.int32)])
        for j in range(D // L):
            vb[r, pl.ds(j * L, L)] = vb[r, pl.ds(j * L, L)] * bc


def _sc_body(x_hbm, rowi_hbm, coli_hbm,
             part_hbm, xs_hbm,
             vb, rb, cb, ones, zrow, dv,
             deg_sh, agg_sh,
             sem):
    c = lax.axis_index("c")
    s = lax.axis_index("s")
    row0 = s * ROWS_PER_TILE

    for g in range(EC // L):
        ones[pl.ds(g * L, L)] = jnp.full((L,), 1.0, jnp.float32)
    for r in range(L):
        for j in range(D // L):
            zrow[r, pl.ds(j * L, L)] = jnp.zeros((L,), jnp.float32)

    def zdv(k, carry):
        dv[pl.ds(k * L, L)] = jnp.zeros((L,), jnp.float32)
        return carry
    lax.fori_loop(0, GROUPS_PER_TILE, zdv, 0)
    pltpu.sync_copy(dv, deg_sh.at[pl.ds(row0, ROWS_PER_TILE)])

    def zagg(k, carry):
        pltpu.sync_copy(zrow, agg_sh.at[pl.ds(row0 + k * L, L)])
        return carry
    lax.fori_loop(0, GROUPS_PER_TILE, zagg, 0)
    plsc.subcore_barrier()

    dbase = s * DEG_PER_TILE

    def deg_step(k, carry):
        pltpu.sync_copy(coli_hbm.at[pl.ds(dbase + k * EC, EC)], cb)
        pltpu.sync_copy(ones, deg_sh.at[cb], add=True)
        return carry
    lax.fori_loop(0, N_DCHUNK, deg_step, 0)
    plsc.subcore_barrier()

    pltpu.sync_copy(deg_sh.at[pl.ds(row0, ROWS_PER_TILE)], dv)

    def dinv_step(k, carry):
        v = dv[pl.ds(k * L, L)]
        y = jnp.where(v >= 0.5, _rsqrt16(v), 0.0)
        dv[pl.ds(k * L, L)] = y
        return carry
    lax.fori_loop(0, GROUPS_PER_TILE, dinv_step, 0)

    def xs_step(k, carry):
        start = row0 + k * L

        @pl.when(start + L <= N_NODES)
        def _():
            pltpu.sync_copy(x_hbm.at[pl.ds(start, L)], vb.at[pl.ds(0, L)])
            _scale_rows(vb, dv, k)
            pltpu.sync_copy(vb.at[pl.ds(0, L)], xs_hbm.at[pl.ds(start, L)])
        return carry
    lax.fori_loop(0, GROUPS_PER_TILE, xs_step, 0)
    plsc.subcore_barrier()

    ebase = (c * NS + s) * E_PER_TILE

    def edge_step(k, carry):
        o = ebase + k * EC
        pltpu.sync_copy(rowi_hbm.at[pl.ds(o, EC)], rb)
        pltpu.sync_copy(coli_hbm.at[pl.ds(o, EC)], cb)
        pltpu.async_copy(xs_hbm.at[rb], vb, sem).wait()
        pltpu.sync_copy(vb, agg_sh.at[cb], add=True)
        return carry
    lax.fori_loop(0, N_ECHUNK, edge_step, 0)
    plsc.subcore_barrier()

    def out_step(k, carry):
        start = row0 + k * L
        pltpu.sync_copy(agg_sh.at[pl.ds(start, L)], vb.at[pl.ds(0, L)])
        _scale_rows(vb, dv, k)
        pltpu.sync_copy(vb.at[pl.ds(0, L)], part_hbm.at[c, pl.ds(start, L)])
        return carry
    lax.fori_loop(0, GROUPS_PER_TILE, out_step, 0)


_sc_kernel = pl.kernel(
    _sc_body,
    out_type=[
        jax.ShapeDtypeStruct((NC, N_PAD, D), jnp.float32),
        jax.ShapeDtypeStruct((N_NODES, D), jnp.float32),
    ],
    mesh=plsc.VectorSubcoreMesh(core_axis_name="c", subcore_axis_name="s"),
    compiler_params=pltpu.CompilerParams(needs_layout_passes=False),
    scratch_types=[
        pltpu.VMEM((EC, D), jnp.float32),
        pltpu.VMEM((EC,), jnp.int32),
        pltpu.VMEM((EC,), jnp.int32),
        pltpu.VMEM((EC,), jnp.float32),
        pltpu.VMEM((L, D), jnp.float32),
        pltpu.VMEM((ROWS_PER_TILE,), jnp.float32),
        pltpu.VMEM_SHARED((N_PAD,), jnp.float32),
        pltpu.VMEM_SHARED((N_PAD, D), jnp.float32),
        pltpu.SemaphoreType.DMA,
    ],
)


def _mm_body(p_ref, wt_ref, b_ref, o_ref):
    a = p_ref[0] + p_ref[1]
    o_ref[...] = (
        jnp.dot(a, wt_ref[...], preferred_element_type=jnp.float32) + b_ref[...]
    )


_BM = 512


def _matmul(parts, wt, b2):
    return pl.pallas_call(
        _mm_body,
        grid=(N_PAD // _BM,),
        in_specs=[
            pl.BlockSpec((NC, _BM, D), lambda i: (0, i, 0)),
            pl.BlockSpec((D, D), lambda i: (0, 0)),
            pl.BlockSpec((1, D), lambda i: (0, 0)),
        ],
        out_specs=pl.BlockSpec((_BM, D), lambda i: (i, 0)),
        out_shape=jax.ShapeDtypeStruct((N_PAD, D), jnp.float32),
    )(parts, wt, b2)


def kernel(x, edge_index, x0, W, b):
    row = edge_index[0].astype(jnp.int32)
    col = edge_index[1].astype(jnp.int32)
    parts, _ = _sc_kernel(x, row, col)
    out = _matmul(parts, W.T, b.reshape(1, D))
    return out[:N_NODES]
